# Initial kernel scaffold; baseline (speedup 1.0000x reference)
import jax
import jax.numpy as jnp
from jax.experimental import pallas as pl


def kernel(x_s, edge_index_s, batch_s, x_t, edge_index_t, batch_t, W1, b1, W2, b2, W3, b3, max_num_nodes):
    raise NotImplementedError("write your pallas kernel here")


# speedup vs baseline: 9.4762x; 9.4762x over previous
import functools

import jax
import jax.numpy as jnp
from jax import lax
from jax.experimental import pallas as pl
from jax.experimental.pallas import tpu as pltpu
from jax.experimental.pallas import tpu_sc as plsc

N = 10000
B = 50
DIN = 128
DH = 128
RESHAPE = 30
MAXN = 256

N_PAD = 10240
CHUNK = 128
NSUB = 16
E_PAD = 32 * 79 * CHUNK
ES = E_PAD // NSUB
NCHUNK = ES // CHUNK
ROWS16 = N_PAD // NSUB

J = 32
NG = 2 * 2 * B * J
GCH = 40
PERW = NG // 32

_mesh = plsc.VectorSubcoreMesh(core_axis_name="c", subcore_axis_name="s")


@functools.partial(
    pl.kernel, mesh=_mesh,
    out_type=jax.ShapeDtypeStruct((2, N_PAD, DH), jnp.float32),
    scratch_types=[
        pltpu.VMEM((CHUNK,), jnp.int32),
        pltpu.VMEM((CHUNK, DH), jnp.float32),
        pltpu.VMEM_SHARED((N_PAD, DH), jnp.float32),
    ],
)
def _sc_deg(dst_hbm, ones_hbm, zeros_hbm, out_hbm, idx_v, ones_v, acc_sh):
    c = lax.axis_index("c")
    s = lax.axis_index("s")
    pltpu.sync_copy(zeros_hbm.at[pl.ds(s * ROWS16, ROWS16)],
                    acc_sh.at[pl.ds(s * ROWS16, ROWS16)])
    pltpu.sync_copy(ones_hbm, ones_v)
    plsc.subcore_barrier()
    base = s * ES

    def body(i, carry):
        off = base + i * CHUNK
        pltpu.sync_copy(dst_hbm.at[c, pl.ds(off, CHUNK)], idx_v)
        pltpu.sync_copy(ones_v, acc_sh.at[idx_v], add=True)
        return carry

    lax.fori_loop(0, NCHUNK, body, 0)
    plsc.subcore_barrier()
    pltpu.sync_copy(acc_sh.at[pl.ds(s * ROWS16, ROWS16)],
                    out_hbm.at[c, pl.ds(s * ROWS16, ROWS16)])


@functools.partial(
    pl.kernel, mesh=_mesh,
    out_type=jax.ShapeDtypeStruct((2, N_PAD, DH), jnp.float32),
    scratch_types=[
        pltpu.VMEM((CHUNK,), jnp.int32),
        pltpu.VMEM((CHUNK,), jnp.int32),
        pltpu.VMEM((CHUNK, DH), jnp.float32),
        pltpu.VMEM_SHARED((N_PAD, DH), jnp.float32),
        pltpu.SemaphoreType.DMA,
    ],
)
def _sc_conv(hp_hbm, srcg_hbm, dst_hbm, zeros_hbm, out_hbm,
             sidx_v, didx_v, rows_v, acc_sh, sem):
    c = lax.axis_index("c")
    s = lax.axis_index("s")
    pltpu.sync_copy(zeros_hbm.at[pl.ds(s * ROWS16, ROWS16)],
                    acc_sh.at[pl.ds(s * ROWS16, ROWS16)])
    plsc.subcore_barrier()
    base = s * ES

    def body(i, carry):
        off = base + i * CHUNK
        pltpu.sync_copy(srcg_hbm.at[c, pl.ds(off, CHUNK)], sidx_v)
        pltpu.sync_copy(dst_hbm.at[c, pl.ds(off, CHUNK)], didx_v)
        pltpu.async_copy(hp_hbm.at[sidx_v], rows_v, sem).wait()
        pltpu.sync_copy(rows_v, acc_sh.at[didx_v], add=True)
        return carry

    lax.fori_loop(0, NCHUNK, body, 0)
    plsc.subcore_barrier()
    pltpu.sync_copy(acc_sh.at[pl.ds(s * ROWS16, ROWS16)],
                    out_hbm.at[c, pl.ds(s * ROWS16, ROWS16)])


@functools.partial(
    pl.kernel, mesh=_mesh,
    out_type=jax.ShapeDtypeStruct((3, NG, DH), jnp.float32),
    scratch_types=[
        pltpu.VMEM((GCH,), jnp.int32),
        pltpu.VMEM((GCH, DH), jnp.float32),
        pltpu.SemaphoreType.DMA,
    ],
)
def _sc_gather(x1_hbm, x2_hbm, x3_hbm, idx_hbm, out_hbm, idx_v, rows_v, sem):
    c = lax.axis_index("c")
    s = lax.axis_index("s")
    w = s * 2 + c
    for t, tab in enumerate((x1_hbm, x2_hbm, x3_hbm)):
        def body(j, carry, tab=tab, t=t):
            off = w * PERW + j * GCH
            pltpu.sync_copy(idx_hbm.at[pl.ds(off, GCH)], idx_v)
            pltpu.async_copy(tab.at[idx_v], rows_v, sem).wait()
            pltpu.sync_copy(rows_v, out_hbm.at[t, pl.ds(off, GCH)])
            return carry

        lax.fori_loop(0, PERW // GCH, body, 0)


_BLK = 256
_GRID = 2 * N_PAD // _BLK


def _mm1_body(x_ref, deg_ref, w_ref, o_ref):
    dinv = lax.rsqrt(deg_ref[:, :1] + 1.0)
    h = jnp.dot(x_ref[...], w_ref[...], preferred_element_type=jnp.float32)
    o_ref[...] = h * dinv


def _mm1(x, deg, W):
    return pl.pallas_call(
        _mm1_body,
        grid=(_GRID,),
        in_specs=[
            pl.BlockSpec((_BLK, DIN), lambda i: (i, 0)),
            pl.BlockSpec((_BLK, DH), lambda i: (i, 0)),
            pl.BlockSpec((DIN, DH), lambda i: (0, 0)),
        ],
        out_specs=pl.BlockSpec((_BLK, DH), lambda i: (i, 0)),
        out_shape=jax.ShapeDtypeStruct((2 * N_PAD, DH), jnp.float32),
    )(x, deg, W)


def _epi_body(agg_ref, hp_ref, deg_ref, b_ref, w_ref, x_ref, hpn_ref):
    dinv = lax.rsqrt(deg_ref[:, :1] + 1.0)
    xl = jnp.maximum(dinv * (agg_ref[...] + hp_ref[...]) + b_ref[...], 0.0)
    x_ref[...] = xl
    hpn_ref[...] = jnp.dot(xl, w_ref[...], preferred_element_type=jnp.float32) * dinv


def _epi(agg, hp, deg, b, Wn):
    return pl.pallas_call(
        _epi_body,
        grid=(_GRID,),
        in_specs=[
            pl.BlockSpec((_BLK, DH), lambda i: (i, 0)),
            pl.BlockSpec((_BLK, DH), lambda i: (i, 0)),
            pl.BlockSpec((_BLK, DH), lambda i: (i, 0)),
            pl.BlockSpec((1, DH), lambda i: (0, 0)),
            pl.BlockSpec((DH, DH), lambda i: (0, 0)),
        ],
        out_specs=[
            pl.BlockSpec((_BLK, DH), lambda i: (i, 0)),
            pl.BlockSpec((_BLK, DH), lambda i: (i, 0)),
        ],
        out_shape=[
            jax.ShapeDtypeStruct((2 * N_PAD, DH), jnp.float32),
            jax.ShapeDtypeStruct((2 * N_PAD, DH), jnp.float32),
        ],
    )(agg, hp, deg, b, Wn)


def _epi3_body(agg_ref, hp_ref, deg_ref, b_ref, x_ref):
    dinv = lax.rsqrt(deg_ref[:, :1] + 1.0)
    x_ref[...] = jnp.maximum(dinv * (agg_ref[...] + hp_ref[...]) + b_ref[...], 0.0)


def _epi3(agg, hp, deg, b):
    return pl.pallas_call(
        _epi3_body,
        grid=(_GRID,),
        in_specs=[
            pl.BlockSpec((_BLK, DH), lambda i: (i, 0)),
            pl.BlockSpec((_BLK, DH), lambda i: (i, 0)),
            pl.BlockSpec((_BLK, DH), lambda i: (i, 0)),
            pl.BlockSpec((1, DH), lambda i: (0, 0)),
        ],
        out_specs=pl.BlockSpec((_BLK, DH), lambda i: (i, 0)),
        out_shape=jax.ShapeDtypeStruct((2 * N_PAD, DH), jnp.float32),
    )(agg, hp, deg, b)


def _bmm_body(a0, a1, b0, b1, w0s, w1s, w0t, w1t, o_ref):
    ys = w0s[0] * a0[0] + w1s[0] * a1[0]
    yt = w0t[0] * b0[0] + w1t[0] * b1[0]
    o_ref[0, 0] = lax.dot_general(ys, yt, (((1,), (1,)), ((), ())),
                                  preferred_element_type=jnp.float32)


def _bmm(a0, a1, b0, b1, w0s, w1s, w0t, w1t):
    row_spec = pl.BlockSpec((1, J, DH), lambda l, b: (l, b, 0))
    wt_spec = pl.BlockSpec((1, J, DH), lambda l, b: (b, 0, 0))
    return pl.pallas_call(
        _bmm_body,
        grid=(3, B),
        in_specs=[row_spec, row_spec, row_spec, row_spec,
                  wt_spec, wt_spec, wt_spec, wt_spec],
        out_specs=pl.BlockSpec((1, 1, J, J), lambda l, b: (b, l, 0, 0)),
        out_shape=jax.ShapeDtypeStruct((B, 3, J, J), jnp.float32),
    )(a0, a1, b0, b1, w0s, w1s, w0t, w1t)


def _pad_edges(ei, core):
    src, dst = ei[0].astype(jnp.int32), ei[1].astype(jnp.int32)
    npad = E_PAD - src.shape[0]
    padr = (N + (jnp.arange(npad, dtype=jnp.int32) % 16))
    src_p = jnp.concatenate([src, padr])
    dst_p = jnp.concatenate([dst, padr])
    return src_p + core * N_PAD, dst_p


def _interp_geometry():
    scale = MAXN / RESHAPE
    c = (jnp.arange(RESHAPE, dtype=jnp.float32) + 0.5) * scale - 0.5
    c0 = jnp.floor(c)
    w = c - c0
    i0 = jnp.clip(c0, 0, MAXN - 1).astype(jnp.int32)
    i1 = jnp.clip(c0 + 1, 0, MAXN - 1).astype(jnp.int32)
    return i0, i1, w


def _proj_idx(batch, max_num_nodes, base, i0, i1, w):
    bd = jnp.searchsorted(batch, jnp.arange(B + 1, dtype=batch.dtype)).astype(jnp.int32)
    starts, counts = bd[:B], bd[1:] - bd[:B]
    lim = jnp.minimum(counts, max_num_nodes)
    v0 = i0[None, :] < lim[:, None]
    v1 = i1[None, :] < lim[:, None]
    spread = (jnp.arange(B, dtype=jnp.int32)[:, None] * J
              + jnp.arange(RESHAPE, dtype=jnp.int32)[None, :])
    g0 = jnp.where(v0, starts[:, None] + i0[None, :], spread) + base
    g1 = jnp.where(v1, starts[:, None] + i1[None, :], spread) + base
    w0 = jnp.where(v0, 1.0 - w[None, :], 0.0)
    w1 = jnp.where(v1, w[None, :], 0.0)
    pad2 = ((0, 0), (0, J - RESHAPE))
    g0 = jnp.pad(g0, pad2, constant_values=base)
    g1 = jnp.pad(g1, pad2, constant_values=base)
    w0 = jnp.pad(w0, pad2)
    w1 = jnp.pad(w1, pad2)
    return (g0.reshape(-1), g1.reshape(-1),
            jnp.broadcast_to(w0.reshape(B, J, 1), (B, J, DH)),
            jnp.broadcast_to(w1.reshape(B, J, 1), (B, J, DH)))


def kernel(x_s, edge_index_s, batch_s, x_t, edge_index_t, batch_t,
           W1, b1, W2, b2, W3, b3, max_num_nodes):
    f32 = jnp.float32
    x_both = jnp.zeros((2 * N_PAD, DIN), f32)
    x_both = x_both.at[:N].set(x_s).at[N_PAD:N_PAD + N].set(x_t)

    src_s, dst_s = _pad_edges(edge_index_s, 0)
    src_t, dst_t = _pad_edges(edge_index_t, 1)
    src_g = jnp.stack([src_s, src_t])
    dst_g = jnp.stack([dst_s, dst_t])

    ones128 = jnp.ones((CHUNK, DH), f32)
    zeros128 = jnp.zeros((N_PAD, DH), f32)

    deg = _sc_deg(dst_g, ones128, zeros128).reshape(2 * N_PAD, DH)

    b1r = b1.reshape(1, DH)
    b2r = b2.reshape(1, DH)
    b3r = b3.reshape(1, DH)

    hp1 = _mm1(x_both, deg, W1)
    agg1 = _sc_conv(hp1, src_g, dst_g, zeros128).reshape(2 * N_PAD, DH)
    x1, hp2 = _epi(agg1, hp1, deg, b1r, W2)
    agg2 = _sc_conv(hp2, src_g, dst_g, zeros128).reshape(2 * N_PAD, DH)
    x2, hp3 = _epi(agg2, hp2, deg, b2r, W3)
    agg3 = _sc_conv(hp3, src_g, dst_g, zeros128).reshape(2 * N_PAD, DH)
    x3 = _epi3(agg3, hp3, deg, b3r)

    i0, i1, w = _interp_geometry()
    g0s, g1s, w0s, w1s = _proj_idx(batch_s, max_num_nodes, 0, i0, i1, w)
    g0t, g1t, w0t, w1t = _proj_idx(batch_t, max_num_nodes, N_PAD, i0, i1, w)
    idx_all = jnp.concatenate([g0s, g1s, g0t, g1t])

    rows = _sc_gather(x1, x2, x3, idx_all)
    nbj = B * J
    a0 = rows[:, 0 * nbj:1 * nbj]
    a1 = rows[:, 1 * nbj:2 * nbj]
    c0 = rows[:, 2 * nbj:3 * nbj]
    c1 = rows[:, 3 * nbj:4 * nbj]

    out_full = _bmm(a0, a1, c0, c1, w0s, w1s, w0t, w1t)
    return out_full[:, :, :RESHAPE, :RESHAPE]


# speedup vs baseline: 15.3247x; 1.6172x over previous
import functools

import jax
import jax.numpy as jnp
from jax import lax
from jax.experimental import pallas as pl
from jax.experimental.pallas import tpu as pltpu
from jax.experimental.pallas import tpu_sc as plsc

N = 10000
B = 50
DIN = 128
DH = 128
RESHAPE = 30
MAXN = 256

N_PAD = 10240
CHUNK = 128
NSUB = 16
E_PAD = 32 * 79 * CHUNK
ES = E_PAD // NSUB
NCHUNK = ES // CHUNK
ROWS16 = N_PAD // NSUB

J = 32
NG = 2 * 2 * B * J
GCH = 40
PERW = NG // 32

_mesh = plsc.VectorSubcoreMesh(core_axis_name="c", subcore_axis_name="s")


@functools.partial(
    pl.kernel, mesh=_mesh,
    out_type=jax.ShapeDtypeStruct((2, N_PAD, DH), jnp.float32),
    scratch_types=[
        pltpu.VMEM((NCHUNK, CHUNK), jnp.int32),
        pltpu.VMEM((CHUNK, DH), jnp.float32),
        pltpu.VMEM_SHARED((N_PAD, DH), jnp.float32),
    ],
)
def _sc_deg(dst_hbm, ones_hbm, zeros_hbm, out_hbm, didx_v, ones_v, acc_sh):
    c = lax.axis_index("c")
    s = lax.axis_index("s")
    pltpu.sync_copy(dst_hbm.at[c, s], didx_v)
    pltpu.sync_copy(zeros_hbm.at[pl.ds(s * ROWS16, ROWS16)],
                    acc_sh.at[pl.ds(s * ROWS16, ROWS16)])
    pltpu.sync_copy(ones_hbm, ones_v)
    plsc.subcore_barrier()

    def body(i, carry):
        pltpu.sync_copy(ones_v, acc_sh.at[didx_v.at[i]], add=True)
        return carry

    lax.fori_loop(0, NCHUNK, body, 0)
    plsc.subcore_barrier()
    pltpu.sync_copy(acc_sh.at[pl.ds(s * ROWS16, ROWS16)],
                    out_hbm.at[c, pl.ds(s * ROWS16, ROWS16)])


@functools.partial(
    pl.kernel, mesh=_mesh,
    out_type=jax.ShapeDtypeStruct((2, N_PAD, DH), jnp.float32),
    scratch_types=[
        pltpu.VMEM((2, CHUNK), jnp.int32),
        pltpu.VMEM((2, CHUNK), jnp.int32),
        pltpu.VMEM((CHUNK, DH), jnp.float32),
        pltpu.VMEM((CHUNK, DH), jnp.float32),
        pltpu.VMEM_SHARED((N_PAD, DH), jnp.float32),
        pltpu.SemaphoreType.DMA,
        pltpu.SemaphoreType.DMA,
        pltpu.SemaphoreType.DMA,
        pltpu.SemaphoreType.DMA,
    ],
)
def _sc_conv(hp_hbm, eidx_hbm, zeros_hbm, out_hbm,
             e0, e1, r0, r1, acc_sh, ise0, ise1, gse0, gse1):
    c = lax.axis_index("c")
    s = lax.axis_index("s")

    def idx_desc(i, ebuf, sem):
        return pltpu.make_async_copy(eidx_hbm.at[c, s, i], ebuf, sem)

    def g_desc(ebuf, rbuf, sem):
        return pltpu.make_async_copy(hp_hbm.at[ebuf.at[0]], rbuf, sem)

    idx_desc(0, e0, ise0).start()
    idx_desc(1, e1, ise1).start()
    pltpu.sync_copy(zeros_hbm.at[pl.ds(s * ROWS16, ROWS16)],
                    acc_sh.at[pl.ds(s * ROWS16, ROWS16)])
    plsc.subcore_barrier()
    idx_desc(0, e0, ise0).wait()
    g_desc(e0, r0, gse0).start()

    def body(i2, carry):
        i = 2 * i2
        g_desc(e0, r0, gse0).wait()
        idx_desc(i + 1, e1, ise1).wait()
        g_desc(e1, r1, gse1).start()
        pltpu.sync_copy(r0, acc_sh.at[e0.at[1]], add=True)

        @pl.when(i + 2 < NCHUNK)
        def _():
            idx_desc(i + 2, e0, ise0).start()

        g_desc(e1, r1, gse1).wait()

        @pl.when(i + 2 < NCHUNK)
        def _():
            idx_desc(i + 2, e0, ise0).wait()
            g_desc(e0, r0, gse0).start()

        pltpu.sync_copy(r1, acc_sh.at[e1.at[1]], add=True)

        @pl.when(i + 3 < NCHUNK)
        def _():
            idx_desc(i + 3, e1, ise1).start()

        return carry

    lax.fori_loop(0, NCHUNK // 2, body, 0)
    plsc.subcore_barrier()
    pltpu.sync_copy(acc_sh.at[pl.ds(s * ROWS16, ROWS16)],
                    out_hbm.at[c, pl.ds(s * ROWS16, ROWS16)])


@functools.partial(
    pl.kernel, mesh=_mesh,
    out_type=jax.ShapeDtypeStruct((3, NG, DH), jnp.float32),
    scratch_types=[
        pltpu.VMEM((PERW,), jnp.int32),
        pltpu.VMEM((GCH, DH), jnp.float32),
        pltpu.VMEM((GCH, DH), jnp.float32),
        pltpu.SemaphoreType.DMA,
        pltpu.SemaphoreType.DMA,
    ],
)
def _sc_gather(x1_hbm, x2_hbm, x3_hbm, idx_hbm, out_hbm,
               idx_v, rows0_v, rows1_v, sem0, sem1):
    c = lax.axis_index("c")
    s = lax.axis_index("s")
    w = s * 2 + c
    pltpu.sync_copy(idx_hbm.at[w], idx_v)
    tabs = (x1_hbm, x2_hbm, x3_hbm)
    chunks = [(t, j) for t in range(3) for j in range(PERW // GCH)]
    bufs = (rows0_v, rows1_v)
    sems = (sem0, sem1)

    def g_desc(k):
        t, j = chunks[k]
        return pltpu.make_async_copy(
            tabs[t].at[idx_v.at[pl.ds(j * GCH, GCH)]], bufs[k % 2], sems[k % 2])

    g_desc(0).start()
    for k, (t, j) in enumerate(chunks):
        g_desc(k).wait()
        if k + 1 < len(chunks):
            g_desc(k + 1).start()
        pltpu.sync_copy(bufs[k % 2],
                        out_hbm.at[t, pl.ds(w * PERW + j * GCH, GCH)])


_BLK = 256
_GRID = 2 * N_PAD // _BLK


def _mm1_body(x_ref, deg_ref, w_ref, o_ref):
    dinv = lax.rsqrt(deg_ref[:, :1] + 1.0)
    h = jnp.dot(x_ref[...], w_ref[...], preferred_element_type=jnp.float32)
    o_ref[...] = h * dinv


def _mm1(x, deg, W):
    return pl.pallas_call(
        _mm1_body,
        grid=(_GRID,),
        in_specs=[
            pl.BlockSpec((_BLK, DIN), lambda i: (i, 0)),
            pl.BlockSpec((_BLK, DH), lambda i: (i, 0)),
            pl.BlockSpec((DIN, DH), lambda i: (0, 0)),
        ],
        out_specs=pl.BlockSpec((_BLK, DH), lambda i: (i, 0)),
        out_shape=jax.ShapeDtypeStruct((2 * N_PAD, DH), jnp.float32),
    )(x, deg, W)


def _epi_body(agg_ref, hp_ref, deg_ref, b_ref, w_ref, x_ref, hpn_ref):
    dinv = lax.rsqrt(deg_ref[:, :1] + 1.0)
    xl = jnp.maximum(dinv * (agg_ref[...] + hp_ref[...]) + b_ref[...], 0.0)
    x_ref[...] = xl
    hpn_ref[...] = jnp.dot(xl, w_ref[...], preferred_element_type=jnp.float32) * dinv


def _epi(agg, hp, deg, b, Wn):
    return pl.pallas_call(
        _epi_body,
        grid=(_GRID,),
        in_specs=[
            pl.BlockSpec((_BLK, DH), lambda i: (i, 0)),
            pl.BlockSpec((_BLK, DH), lambda i: (i, 0)),
            pl.BlockSpec((_BLK, DH), lambda i: (i, 0)),
            pl.BlockSpec((1, DH), lambda i: (0, 0)),
            pl.BlockSpec((DH, DH), lambda i: (0, 0)),
        ],
        out_specs=[
            pl.BlockSpec((_BLK, DH), lambda i: (i, 0)),
            pl.BlockSpec((_BLK, DH), lambda i: (i, 0)),
        ],
        out_shape=[
            jax.ShapeDtypeStruct((2 * N_PAD, DH), jnp.float32),
            jax.ShapeDtypeStruct((2 * N_PAD, DH), jnp.float32),
        ],
    )(agg, hp, deg, b, Wn)


def _epi3_body(agg_ref, hp_ref, deg_ref, b_ref, x_ref):
    dinv = lax.rsqrt(deg_ref[:, :1] + 1.0)
    x_ref[...] = jnp.maximum(dinv * (agg_ref[...] + hp_ref[...]) + b_ref[...], 0.0)


def _epi3(agg, hp, deg, b):
    return pl.pallas_call(
        _epi3_body,
        grid=(_GRID,),
        in_specs=[
            pl.BlockSpec((_BLK, DH), lambda i: (i, 0)),
            pl.BlockSpec((_BLK, DH), lambda i: (i, 0)),
            pl.BlockSpec((_BLK, DH), lambda i: (i, 0)),
            pl.BlockSpec((1, DH), lambda i: (0, 0)),
        ],
        out_specs=pl.BlockSpec((_BLK, DH), lambda i: (i, 0)),
        out_shape=jax.ShapeDtypeStruct((2 * N_PAD, DH), jnp.float32),
    )(agg, hp, deg, b)


def _bmm_body(a0, a1, b0, b1, w0s, w1s, w0t, w1t, o_ref):
    ys = w0s[0] * a0[0] + w1s[0] * a1[0]
    yt = w0t[0] * b0[0] + w1t[0] * b1[0]
    o_ref[0, 0] = lax.dot_general(ys, yt, (((1,), (1,)), ((), ())),
                                  preferred_element_type=jnp.float32)


def _bmm(a0, a1, b0, b1, w0s, w1s, w0t, w1t):
    row_spec = pl.BlockSpec((1, J, DH), lambda l, b: (l, b, 0))
    wt_spec = pl.BlockSpec((1, J, DH), lambda l, b: (b, 0, 0))
    return pl.pallas_call(
        _bmm_body,
        grid=(3, B),
        in_specs=[row_spec, row_spec, row_spec, row_spec,
                  wt_spec, wt_spec, wt_spec, wt_spec],
        out_specs=pl.BlockSpec((1, 1, J, J), lambda l, b: (b, l, 0, 0)),
        out_shape=jax.ShapeDtypeStruct((B, 3, J, J), jnp.float32),
    )(a0, a1, b0, b1, w0s, w1s, w0t, w1t)


def _pad_edges(ei, core):
    src, dst = ei[0].astype(jnp.int32), ei[1].astype(jnp.int32)
    npad = E_PAD - src.shape[0]
    padr = (N + (jnp.arange(npad, dtype=jnp.int32) % 16))
    src_p = jnp.concatenate([src, padr])
    dst_p = jnp.concatenate([dst, padr])
    return src_p + core * N_PAD, dst_p


def _interp_geometry():
    scale = MAXN / RESHAPE
    c = (jnp.arange(RESHAPE, dtype=jnp.float32) + 0.5) * scale - 0.5
    c0 = jnp.floor(c)
    w = c - c0
    i0 = jnp.clip(c0, 0, MAXN - 1).astype(jnp.int32)
    i1 = jnp.clip(c0 + 1, 0, MAXN - 1).astype(jnp.int32)
    return i0, i1, w


def _proj_idx(batch, max_num_nodes, base, i0, i1, w):
    bd = jnp.searchsorted(batch, jnp.arange(B + 1, dtype=batch.dtype)).astype(jnp.int32)
    starts, counts = bd[:B], bd[1:] - bd[:B]
    lim = jnp.minimum(counts, max_num_nodes)
    v0 = i0[None, :] < lim[:, None]
    v1 = i1[None, :] < lim[:, None]
    spread = (jnp.arange(B, dtype=jnp.int32)[:, None] * J
              + jnp.arange(RESHAPE, dtype=jnp.int32)[None, :])
    g0 = jnp.where(v0, starts[:, None] + i0[None, :], spread) + base
    g1 = jnp.where(v1, starts[:, None] + i1[None, :], spread) + base
    w0 = jnp.where(v0, 1.0 - w[None, :], 0.0)
    w1 = jnp.where(v1, w[None, :], 0.0)
    pad2 = ((0, 0), (0, J - RESHAPE))
    g0 = jnp.pad(g0, pad2, constant_values=base)
    g1 = jnp.pad(g1, pad2, constant_values=base)
    w0 = jnp.pad(w0, pad2)
    w1 = jnp.pad(w1, pad2)
    return (g0.reshape(-1), g1.reshape(-1),
            jnp.broadcast_to(w0.reshape(B, J, 1), (B, J, DH)),
            jnp.broadcast_to(w1.reshape(B, J, 1), (B, J, DH)))


def kernel(x_s, edge_index_s, batch_s, x_t, edge_index_t, batch_t,
           W1, b1, W2, b2, W3, b3, max_num_nodes):
    f32 = jnp.float32
    x_both = jnp.zeros((2 * N_PAD, DIN), f32)
    x_both = x_both.at[:N].set(x_s).at[N_PAD:N_PAD + N].set(x_t)

    src_s, dst_s = _pad_edges(edge_index_s, 0)
    src_t, dst_t = _pad_edges(edge_index_t, 1)
    src_a = jnp.stack([src_s, src_t]).reshape(2, NSUB, NCHUNK, 1, CHUNK)
    dst_a = jnp.stack([dst_s, dst_t]).reshape(2, NSUB, NCHUNK, 1, CHUNK)
    eidx = jnp.concatenate([src_a, dst_a], axis=3)
    dst_g = dst_a.reshape(2, NSUB, NCHUNK, CHUNK)

    ones128 = jnp.ones((CHUNK, DH), f32)
    zeros128 = jnp.zeros((N_PAD, DH), f32)

    deg = _sc_deg(dst_g, ones128, zeros128).reshape(2 * N_PAD, DH)

    b1r = b1.reshape(1, DH)
    b2r = b2.reshape(1, DH)
    b3r = b3.reshape(1, DH)

    hp1 = _mm1(x_both, deg, W1)
    agg1 = _sc_conv(hp1, eidx, zeros128).reshape(2 * N_PAD, DH)
    x1, hp2 = _epi(agg1, hp1, deg, b1r, W2)
    agg2 = _sc_conv(hp2, eidx, zeros128).reshape(2 * N_PAD, DH)
    x2, hp3 = _epi(agg2, hp2, deg, b2r, W3)
    agg3 = _sc_conv(hp3, eidx, zeros128).reshape(2 * N_PAD, DH)
    x3 = _epi3(agg3, hp3, deg, b3r)

    i0, i1, w = _interp_geometry()
    g0s, g1s, w0s, w1s = _proj_idx(batch_s, max_num_nodes, 0, i0, i1, w)
    g0t, g1t, w0t, w1t = _proj_idx(batch_t, max_num_nodes, N_PAD, i0, i1, w)
    idx_all = jnp.concatenate([g0s, g1s, g0t, g1t]).reshape(32, PERW)

    rows = _sc_gather(x1, x2, x3, idx_all)
    nbj = B * J
    a0 = rows[:, 0 * nbj:1 * nbj]
    a1 = rows[:, 1 * nbj:2 * nbj]
    c0 = rows[:, 2 * nbj:3 * nbj]
    c1 = rows[:, 3 * nbj:4 * nbj]

    out_full = _bmm(a0, a1, c0, c1, w0s, w1s, w0t, w1t)
    return out_full[:, :, :RESHAPE, :RESHAPE]


# speedup vs baseline: 15.4321x; 1.0070x over previous
import functools

import jax
import jax.numpy as jnp
from jax import lax
from jax.experimental import pallas as pl
from jax.experimental.pallas import tpu as pltpu
from jax.experimental.pallas import tpu_sc as plsc

N = 10000
B = 50
DIN = 128
DH = 128
RESHAPE = 30
MAXN = 256

N_PAD = 10240
CHUNK = 128
NSUB = 16
E_PAD = 32 * 79 * CHUNK
ES = E_PAD // NSUB
NCHUNK = ES // CHUNK
ROWS16 = N_PAD // NSUB

J = 32
NG = 2 * 2 * B * J
GCH = 128
PERW = NG // 32
GCHUNKS = ((0, 128), (128, 72))

_mesh = plsc.VectorSubcoreMesh(core_axis_name="c", subcore_axis_name="s")


@functools.partial(
    pl.kernel, mesh=_mesh,
    out_type=jax.ShapeDtypeStruct((2, N_PAD, DH), jnp.float32),
    scratch_types=[
        pltpu.VMEM((NCHUNK, CHUNK), jnp.int32),
        pltpu.VMEM((CHUNK, DH), jnp.float32),
        pltpu.VMEM_SHARED((N_PAD, DH), jnp.float32),
    ],
)
def _sc_deg(dst_hbm, ones_hbm, zeros_hbm, out_hbm, didx_v, ones_v, acc_sh):
    c = lax.axis_index("c")
    s = lax.axis_index("s")
    pltpu.sync_copy(dst_hbm.at[c, s], didx_v)
    pltpu.sync_copy(zeros_hbm.at[pl.ds(s * ROWS16, ROWS16)],
                    acc_sh.at[pl.ds(s * ROWS16, ROWS16)])
    pltpu.sync_copy(ones_hbm, ones_v)
    plsc.subcore_barrier()

    def body(i, carry):
        pltpu.sync_copy(ones_v, acc_sh.at[didx_v.at[i]], add=True)
        return carry

    lax.fori_loop(0, NCHUNK, body, 0)
    plsc.subcore_barrier()
    pltpu.sync_copy(acc_sh.at[pl.ds(s * ROWS16, ROWS16)],
                    out_hbm.at[c, pl.ds(s * ROWS16, ROWS16)])


@functools.partial(
    pl.kernel, mesh=_mesh,
    out_type=jax.ShapeDtypeStruct((2, N_PAD, DH), jnp.float32),
    scratch_types=[
        pltpu.VMEM((2, CHUNK), jnp.int32),
        pltpu.VMEM((2, CHUNK), jnp.int32),
        pltpu.VMEM((CHUNK, DH), jnp.float32),
        pltpu.VMEM((CHUNK, DH), jnp.float32),
        pltpu.VMEM_SHARED((N_PAD, DH), jnp.float32),
        pltpu.SemaphoreType.DMA,
        pltpu.SemaphoreType.DMA,
        pltpu.SemaphoreType.DMA,
        pltpu.SemaphoreType.DMA,
    ],
)
def _sc_conv(hp_hbm, eidx_hbm, zeros_hbm, out_hbm,
             e0, e1, r0, r1, acc_sh, ise0, ise1, gse0, gse1):
    c = lax.axis_index("c")
    s = lax.axis_index("s")

    def idx_desc(i, ebuf, sem):
        return pltpu.make_async_copy(eidx_hbm.at[c, s, i], ebuf, sem)

    def g_desc(ebuf, rbuf, sem):
        return pltpu.make_async_copy(hp_hbm.at[ebuf.at[0]], rbuf, sem)

    idx_desc(0, e0, ise0).start()
    idx_desc(1, e1, ise1).start()
    pltpu.sync_copy(zeros_hbm.at[pl.ds(s * ROWS16, ROWS16)],
                    acc_sh.at[pl.ds(s * ROWS16, ROWS16)])
    plsc.subcore_barrier()
    idx_desc(0, e0, ise0).wait()
    g_desc(e0, r0, gse0).start()

    def body(i2, carry):
        i = 2 * i2
        g_desc(e0, r0, gse0).wait()
        idx_desc(i + 1, e1, ise1).wait()
        g_desc(e1, r1, gse1).start()
        pltpu.sync_copy(r0, acc_sh.at[e0.at[1]], add=True)

        @pl.when(i + 2 < NCHUNK)
        def _():
            idx_desc(i + 2, e0, ise0).start()

        g_desc(e1, r1, gse1).wait()

        @pl.when(i + 2 < NCHUNK)
        def _():
            idx_desc(i + 2, e0, ise0).wait()
            g_desc(e0, r0, gse0).start()

        pltpu.sync_copy(r1, acc_sh.at[e1.at[1]], add=True)

        @pl.when(i + 3 < NCHUNK)
        def _():
            idx_desc(i + 3, e1, ise1).start()

        return carry

    lax.fori_loop(0, NCHUNK // 2, body, 0)
    plsc.subcore_barrier()
    pltpu.sync_copy(acc_sh.at[pl.ds(s * ROWS16, ROWS16)],
                    out_hbm.at[c, pl.ds(s * ROWS16, ROWS16)])


@functools.partial(
    pl.kernel, mesh=_mesh,
    out_type=jax.ShapeDtypeStruct((3, NG, DH), jnp.float32),
    scratch_types=[
        pltpu.VMEM((PERW,), jnp.int32),
        pltpu.VMEM((GCH, DH), jnp.float32),
        pltpu.VMEM((GCH, DH), jnp.float32),
        pltpu.SemaphoreType.DMA,
        pltpu.SemaphoreType.DMA,
    ],
)
def _sc_gather(x1_hbm, x2_hbm, x3_hbm, idx_hbm, out_hbm,
               idx_v, rows0_v, rows1_v, sem0, sem1):
    c = lax.axis_index("c")
    s = lax.axis_index("s")
    w = s * 2 + c
    pltpu.sync_copy(idx_hbm.at[w], idx_v)
    tabs = (x1_hbm, x2_hbm, x3_hbm)
    chunks = [(t, off, sz) for t in range(3) for (off, sz) in GCHUNKS]
    bufs = (rows0_v, rows1_v)
    sems = (sem0, sem1)

    def g_desc(k):
        t, off, sz = chunks[k]
        return pltpu.make_async_copy(
            tabs[t].at[idx_v.at[pl.ds(off, sz)]],
            bufs[k % 2].at[pl.ds(0, sz)], sems[k % 2])

    g_desc(0).start()
    for k, (t, off, sz) in enumerate(chunks):
        g_desc(k).wait()
        if k + 1 < len(chunks):
            g_desc(k + 1).start()
        pltpu.sync_copy(bufs[k % 2].at[pl.ds(0, sz)],
                        out_hbm.at[t, pl.ds(w * PERW + off, sz)])


_BLK = 256
_GRID = 2 * N_PAD // _BLK


def _mmh_body(x_ref, w_ref, o_ref):
    o_ref[...] = jnp.dot(x_ref[...], w_ref[...], preferred_element_type=jnp.float32)


def _mmh(x, W):
    return pl.pallas_call(
        _mmh_body,
        grid=(_GRID,),
        in_specs=[
            pl.BlockSpec((_BLK, DIN), lambda i: (i, 0)),
            pl.BlockSpec((DIN, DH), lambda i: (0, 0)),
        ],
        out_specs=pl.BlockSpec((_BLK, DH), lambda i: (i, 0)),
        out_shape=jax.ShapeDtypeStruct((2 * N_PAD, DH), jnp.float32),
    )(x, W)


def _scale_body(h_ref, deg_ref, o_ref):
    o_ref[...] = h_ref[...] * lax.rsqrt(deg_ref[:, :1] + 1.0)


def _scale(h, deg):
    return pl.pallas_call(
        _scale_body,
        grid=(_GRID,),
        in_specs=[
            pl.BlockSpec((_BLK, DH), lambda i: (i, 0)),
            pl.BlockSpec((_BLK, DH), lambda i: (i, 0)),
        ],
        out_specs=pl.BlockSpec((_BLK, DH), lambda i: (i, 0)),
        out_shape=jax.ShapeDtypeStruct((2 * N_PAD, DH), jnp.float32),
    )(h, deg)


def _epi_body(agg_ref, hp_ref, deg_ref, b_ref, w_ref, x_ref, hpn_ref):
    dinv = lax.rsqrt(deg_ref[:, :1] + 1.0)
    xl = jnp.maximum(dinv * (agg_ref[...] + hp_ref[...]) + b_ref[...], 0.0)
    x_ref[...] = xl
    hpn_ref[...] = jnp.dot(xl, w_ref[...], preferred_element_type=jnp.float32) * dinv


def _epi(agg, hp, deg, b, Wn):
    return pl.pallas_call(
        _epi_body,
        grid=(_GRID,),
        in_specs=[
            pl.BlockSpec((_BLK, DH), lambda i: (i, 0)),
            pl.BlockSpec((_BLK, DH), lambda i: (i, 0)),
            pl.BlockSpec((_BLK, DH), lambda i: (i, 0)),
            pl.BlockSpec((1, DH), lambda i: (0, 0)),
            pl.BlockSpec((DH, DH), lambda i: (0, 0)),
        ],
        out_specs=[
            pl.BlockSpec((_BLK, DH), lambda i: (i, 0)),
            pl.BlockSpec((_BLK, DH), lambda i: (i, 0)),
        ],
        out_shape=[
            jax.ShapeDtypeStruct((2 * N_PAD, DH), jnp.float32),
            jax.ShapeDtypeStruct((2 * N_PAD, DH), jnp.float32),
        ],
    )(agg, hp, deg, b, Wn)


def _epi3_body(agg_ref, hp_ref, deg_ref, b_ref, x_ref):
    dinv = lax.rsqrt(deg_ref[:, :1] + 1.0)
    x_ref[...] = jnp.maximum(dinv * (agg_ref[...] + hp_ref[...]) + b_ref[...], 0.0)


def _epi3(agg, hp, deg, b):
    return pl.pallas_call(
        _epi3_body,
        grid=(_GRID,),
        in_specs=[
            pl.BlockSpec((_BLK, DH), lambda i: (i, 0)),
            pl.BlockSpec((_BLK, DH), lambda i: (i, 0)),
            pl.BlockSpec((_BLK, DH), lambda i: (i, 0)),
            pl.BlockSpec((1, DH), lambda i: (0, 0)),
        ],
        out_specs=pl.BlockSpec((_BLK, DH), lambda i: (i, 0)),
        out_shape=jax.ShapeDtypeStruct((2 * N_PAD, DH), jnp.float32),
    )(agg, hp, deg, b)


def _bmm_body(a0, a1, b0, b1, w0s, w1s, w0t, w1t, o_ref):
    ys = w0s[0] * a0[0] + w1s[0] * a1[0]
    yt = w0t[0] * b0[0] + w1t[0] * b1[0]
    o_ref[0, 0] = lax.dot_general(ys, yt, (((1,), (1,)), ((), ())),
                                  preferred_element_type=jnp.float32)


def _bmm(a0, a1, b0, b1, w0s, w1s, w0t, w1t):
    row_spec = pl.BlockSpec((1, J, DH), lambda l, b: (l, b, 0))
    wt_spec = pl.BlockSpec((1, J, DH), lambda l, b: (b, 0, 0))
    return pl.pallas_call(
        _bmm_body,
        grid=(3, B),
        in_specs=[row_spec, row_spec, row_spec, row_spec,
                  wt_spec, wt_spec, wt_spec, wt_spec],
        out_specs=pl.BlockSpec((1, 1, J, J), lambda l, b: (b, l, 0, 0)),
        out_shape=jax.ShapeDtypeStruct((B, 3, J, J), jnp.float32),
    )(a0, a1, b0, b1, w0s, w1s, w0t, w1t)


def _pad_edges(ei, core):
    src, dst = ei[0].astype(jnp.int32), ei[1].astype(jnp.int32)
    npad = E_PAD - src.shape[0]
    padr = (N + (jnp.arange(npad, dtype=jnp.int32) % 16))
    src_p = jnp.concatenate([src, padr])
    dst_p = jnp.concatenate([dst, padr])
    return src_p + core * N_PAD, dst_p


def _interp_geometry():
    scale = MAXN / RESHAPE
    c = (jnp.arange(RESHAPE, dtype=jnp.float32) + 0.5) * scale - 0.5
    c0 = jnp.floor(c)
    w = c - c0
    i0 = jnp.clip(c0, 0, MAXN - 1).astype(jnp.int32)
    i1 = jnp.clip(c0 + 1, 0, MAXN - 1).astype(jnp.int32)
    return i0, i1, w


def _proj_idx(batch, max_num_nodes, base, i0, i1, w):
    bd = jnp.searchsorted(batch, jnp.arange(B + 1, dtype=batch.dtype)).astype(jnp.int32)
    starts, counts = bd[:B], bd[1:] - bd[:B]
    lim = jnp.minimum(counts, max_num_nodes)
    v0 = i0[None, :] < lim[:, None]
    v1 = i1[None, :] < lim[:, None]
    spread = (jnp.arange(B, dtype=jnp.int32)[:, None] * J
              + jnp.arange(RESHAPE, dtype=jnp.int32)[None, :])
    g0 = jnp.where(v0, starts[:, None] + i0[None, :], spread) + base
    g1 = jnp.where(v1, starts[:, None] + i1[None, :], spread) + base
    w0 = jnp.where(v0, 1.0 - w[None, :], 0.0)
    w1 = jnp.where(v1, w[None, :], 0.0)
    pad2 = ((0, 0), (0, J - RESHAPE))
    g0 = jnp.pad(g0, pad2, constant_values=base)
    g1 = jnp.pad(g1, pad2, constant_values=base)
    w0 = jnp.pad(w0, pad2)
    w1 = jnp.pad(w1, pad2)
    return (g0.reshape(-1), g1.reshape(-1),
            jnp.broadcast_to(w0.reshape(B, J, 1), (B, J, DH)),
            jnp.broadcast_to(w1.reshape(B, J, 1), (B, J, DH)))


def kernel(x_s, edge_index_s, batch_s, x_t, edge_index_t, batch_t,
           W1, b1, W2, b2, W3, b3, max_num_nodes):
    f32 = jnp.float32
    x_both = jnp.zeros((2 * N_PAD, DIN), f32)
    x_both = x_both.at[:N].set(x_s).at[N_PAD:N_PAD + N].set(x_t)

    src_s, dst_s = _pad_edges(edge_index_s, 0)
    src_t, dst_t = _pad_edges(edge_index_t, 1)
    src_a = jnp.stack([src_s, src_t]).reshape(2, NSUB, NCHUNK, 1, CHUNK)
    dst_a = jnp.stack([dst_s, dst_t]).reshape(2, NSUB, NCHUNK, 1, CHUNK)
    eidx = jnp.concatenate([src_a, dst_a], axis=3)
    dst_g = dst_a.reshape(2, NSUB, NCHUNK, CHUNK)

    ones128 = jnp.ones((CHUNK, DH), f32)
    zeros128 = jnp.zeros((N_PAD, DH), f32)

    h1 = _mmh(x_both, W1)
    deg = _sc_deg(dst_g, ones128, zeros128).reshape(2 * N_PAD, DH)

    b1r = b1.reshape(1, DH)
    b2r = b2.reshape(1, DH)
    b3r = b3.reshape(1, DH)

    hp1 = _scale(h1, deg)
    agg1 = _sc_conv(hp1, eidx, zeros128).reshape(2 * N_PAD, DH)
    x1, hp2 = _epi(agg1, hp1, deg, b1r, W2)
    agg2 = _sc_conv(hp2, eidx, zeros128).reshape(2 * N_PAD, DH)
    x2, hp3 = _epi(agg2, hp2, deg, b2r, W3)
    agg3 = _sc_conv(hp3, eidx, zeros128).reshape(2 * N_PAD, DH)
    x3 = _epi3(agg3, hp3, deg, b3r)

    i0, i1, w = _interp_geometry()
    g0s, g1s, w0s, w1s = _proj_idx(batch_s, max_num_nodes, 0, i0, i1, w)
    g0t, g1t, w0t, w1t = _proj_idx(batch_t, max_num_nodes, N_PAD, i0, i1, w)
    idx_all = jnp.concatenate([g0s, g1s, g0t, g1t]).reshape(32, PERW)

    rows = _sc_gather(x1, x2, x3, idx_all)
    nbj = B * J
    a0 = rows[:, 0 * nbj:1 * nbj]
    a1 = rows[:, 1 * nbj:2 * nbj]
    c0 = rows[:, 2 * nbj:3 * nbj]
    c1 = rows[:, 3 * nbj:4 * nbj]

    out_full = _bmm(a0, a1, c0, c1, w0s, w1s, w0t, w1t)
    return out_full[:, :, :RESHAPE, :RESHAPE]


# speedup vs baseline: 15.6423x; 1.0136x over previous
import functools

import jax
import jax.numpy as jnp
from jax import lax
from jax.experimental import pallas as pl
from jax.experimental.pallas import tpu as pltpu
from jax.experimental.pallas import tpu_sc as plsc

N = 10000
B = 50
DIN = 128
DH = 128
RESHAPE = 30
MAXN = 256

N_PAD = 10240
CHUNK = 128
NSUB = 16
E_PAD = 32 * 79 * CHUNK
ES = E_PAD // NSUB
NCHUNK = ES // CHUNK
ROWS16 = N_PAD // NSUB

J = 32
NG = 2 * 2 * B * J
GCH = 128
PERW = NG // 32
GCHUNKS = ((0, 128), (128, 72))

_mesh = plsc.VectorSubcoreMesh(core_axis_name="c", subcore_axis_name="s")


@functools.partial(
    pl.kernel, mesh=_mesh,
    out_type=jax.ShapeDtypeStruct((2, N_PAD, DH), jnp.float32),
    scratch_types=[
        pltpu.VMEM((NCHUNK, CHUNK), jnp.int32),
        pltpu.VMEM((CHUNK, DH), jnp.float32),
        pltpu.VMEM_SHARED((N_PAD, DH), jnp.float32),
    ],
)
def _sc_deg(dst_hbm, ones_hbm, zeros_hbm, out_hbm, didx_v, ones_v, acc_sh):
    c = lax.axis_index("c")
    s = lax.axis_index("s")
    pltpu.sync_copy(dst_hbm.at[c, s], didx_v)
    pltpu.sync_copy(zeros_hbm.at[pl.ds(s * ROWS16, ROWS16)],
                    acc_sh.at[pl.ds(s * ROWS16, ROWS16)])
    pltpu.sync_copy(ones_hbm, ones_v)
    plsc.subcore_barrier()

    def body(i, carry):
        pltpu.sync_copy(ones_v, acc_sh.at[didx_v.at[i]], add=True)
        return carry

    lax.fori_loop(0, NCHUNK, body, 0)
    plsc.subcore_barrier()
    pltpu.sync_copy(acc_sh.at[pl.ds(s * ROWS16, ROWS16)],
                    out_hbm.at[c, pl.ds(s * ROWS16, ROWS16)])


@functools.partial(
    pl.kernel, mesh=_mesh,
    out_type=jax.ShapeDtypeStruct((2, N_PAD, DH), jnp.float32),
    scratch_types=[
        pltpu.VMEM((2, CHUNK), jnp.int32),
        pltpu.VMEM((2, CHUNK), jnp.int32),
        pltpu.VMEM((CHUNK, DH), jnp.float32),
        pltpu.VMEM((CHUNK, DH), jnp.float32),
        pltpu.VMEM_SHARED((N_PAD, DH), jnp.float32),
        pltpu.SemaphoreType.DMA,
        pltpu.SemaphoreType.DMA,
        pltpu.SemaphoreType.DMA,
        pltpu.SemaphoreType.DMA,
    ],
)
def _sc_conv(hp_hbm, eidx_hbm, zeros_hbm, out_hbm,
             e0, e1, r0, r1, acc_sh, ise0, ise1, gse0, gse1):
    c = lax.axis_index("c")
    s = lax.axis_index("s")

    def idx_desc(i, ebuf, sem):
        return pltpu.make_async_copy(eidx_hbm.at[c, s, i], ebuf, sem)

    def g_desc(ebuf, rbuf, sem):
        return pltpu.make_async_copy(hp_hbm.at[ebuf.at[0]], rbuf, sem)

    idx_desc(0, e0, ise0).start()
    idx_desc(1, e1, ise1).start()
    pltpu.sync_copy(zeros_hbm.at[pl.ds(s * ROWS16, ROWS16)],
                    acc_sh.at[pl.ds(s * ROWS16, ROWS16)])
    plsc.subcore_barrier()
    idx_desc(0, e0, ise0).wait()
    g_desc(e0, r0, gse0).start()

    def body(i2, carry):
        i = 2 * i2
        g_desc(e0, r0, gse0).wait()
        idx_desc(i + 1, e1, ise1).wait()
        g_desc(e1, r1, gse1).start()
        pltpu.sync_copy(r0, acc_sh.at[e0.at[1]], add=True)

        @pl.when(i + 2 < NCHUNK)
        def _():
            idx_desc(i + 2, e0, ise0).start()

        g_desc(e1, r1, gse1).wait()

        @pl.when(i + 2 < NCHUNK)
        def _():
            idx_desc(i + 2, e0, ise0).wait()
            g_desc(e0, r0, gse0).start()

        pltpu.sync_copy(r1, acc_sh.at[e1.at[1]], add=True)

        @pl.when(i + 3 < NCHUNK)
        def _():
            idx_desc(i + 3, e1, ise1).start()

        return carry

    lax.fori_loop(0, NCHUNK // 2, body, 0)
    plsc.subcore_barrier()
    pltpu.sync_copy(acc_sh.at[pl.ds(s * ROWS16, ROWS16)],
                    out_hbm.at[c, pl.ds(s * ROWS16, ROWS16)])


@functools.partial(
    pl.kernel, mesh=_mesh,
    out_type=jax.ShapeDtypeStruct((5, NG, DH), jnp.float32),
    scratch_types=[
        pltpu.VMEM((2, CHUNK), jnp.int32),
        pltpu.VMEM((2, CHUNK), jnp.int32),
        pltpu.VMEM((CHUNK, DH), jnp.float32),
        pltpu.VMEM((CHUNK, DH), jnp.float32),
        pltpu.VMEM((PERW,), jnp.int32),
        pltpu.VMEM((PERW,), jnp.int32),
        pltpu.VMEM_SHARED((N_PAD, DH), jnp.float32),
        pltpu.SemaphoreType.DMA,
        pltpu.SemaphoreType.DMA,
        pltpu.SemaphoreType.DMA,
        pltpu.SemaphoreType.DMA,
    ],
)
def _sc_conv3(hp_hbm, eidx_hbm, zeros_hbm, x1_hbm, x2_hbm, deg_hbm,
              gidx_hbm, aggidx_hbm, rows_out,
              e0, e1, r0, r1, gi_v, ai_v, acc_sh, ise0, ise1, gse0, gse1):
    c = lax.axis_index("c")
    s = lax.axis_index("s")

    def idx_desc(i, ebuf, sem):
        return pltpu.make_async_copy(eidx_hbm.at[c, s, i], ebuf, sem)

    def g_desc(ebuf, rbuf, sem):
        return pltpu.make_async_copy(hp_hbm.at[ebuf.at[0]], rbuf, sem)

    idx_desc(0, e0, ise0).start()
    idx_desc(1, e1, ise1).start()
    pltpu.sync_copy(zeros_hbm.at[pl.ds(s * ROWS16, ROWS16)],
                    acc_sh.at[pl.ds(s * ROWS16, ROWS16)])
    plsc.subcore_barrier()
    idx_desc(0, e0, ise0).wait()
    g_desc(e0, r0, gse0).start()

    def body(i2, carry):
        i = 2 * i2
        g_desc(e0, r0, gse0).wait()
        idx_desc(i + 1, e1, ise1).wait()
        g_desc(e1, r1, gse1).start()
        pltpu.sync_copy(r0, acc_sh.at[e0.at[1]], add=True)

        @pl.when(i + 2 < NCHUNK)
        def _():
            idx_desc(i + 2, e0, ise0).start()

        g_desc(e1, r1, gse1).wait()

        @pl.when(i + 2 < NCHUNK)
        def _():
            idx_desc(i + 2, e0, ise0).wait()
            g_desc(e0, r0, gse0).start()

        pltpu.sync_copy(r1, acc_sh.at[e1.at[1]], add=True)

        @pl.when(i + 3 < NCHUNK)
        def _():
            idx_desc(i + 3, e1, ise1).start()

        return carry

    lax.fori_loop(0, NCHUNK // 2, body, 0)
    plsc.subcore_barrier()

    w2 = c * NSUB + s
    pltpu.sync_copy(gidx_hbm.at[w2], gi_v)
    pltpu.sync_copy(aggidx_hbm.at[w2], ai_v)
    jobs = ([(t, off, sz, t, False) for t, _ in enumerate((0, 1, 2, 3))
             for (off, sz) in GCHUNKS]
            + [(4, off, sz, 0, True) for (off, sz) in GCHUNKS])
    tabs = (x1_hbm, x2_hbm, hp_hbm, deg_hbm)
    bufs = (r0, r1)
    sems = (gse0, gse1)

    def j_desc(k):
        t, off, sz, ti, from_acc = jobs[k]
        src = acc_sh if from_acc else tabs[ti]
        iv = ai_v if from_acc else gi_v
        return pltpu.make_async_copy(
            src.at[iv.at[pl.ds(off, sz)]],
            bufs[k % 2].at[pl.ds(0, sz)], sems[k % 2])

    j_desc(0).start()
    for k, (t, off, sz, ti, from_acc) in enumerate(jobs):
        j_desc(k).wait()
        if k + 1 < len(jobs):
            j_desc(k + 1).start()
        pltpu.sync_copy(bufs[k % 2].at[pl.ds(0, sz)],
                        rows_out.at[t, pl.ds(w2 * PERW + off, sz)])


_BLK = 256
_GRID = 2 * N_PAD // _BLK


def _mmh_body(x_ref, w_ref, o_ref):
    o_ref[...] = jnp.dot(x_ref[...], w_ref[...], preferred_element_type=jnp.float32)


def _mmh(x, W):
    return pl.pallas_call(
        _mmh_body,
        grid=(_GRID,),
        in_specs=[
            pl.BlockSpec((_BLK, DIN), lambda i: (i, 0)),
            pl.BlockSpec((DIN, DH), lambda i: (0, 0)),
        ],
        out_specs=pl.BlockSpec((_BLK, DH), lambda i: (i, 0)),
        out_shape=jax.ShapeDtypeStruct((2 * N_PAD, DH), jnp.float32),
    )(x, W)


def _scale_body(h_ref, deg_ref, o_ref):
    o_ref[...] = h_ref[...] * lax.rsqrt(deg_ref[:, :1] + 1.0)


def _scale(h, deg):
    return pl.pallas_call(
        _scale_body,
        grid=(_GRID,),
        in_specs=[
            pl.BlockSpec((_BLK, DH), lambda i: (i, 0)),
            pl.BlockSpec((_BLK, DH), lambda i: (i, 0)),
        ],
        out_specs=pl.BlockSpec((_BLK, DH), lambda i: (i, 0)),
        out_shape=jax.ShapeDtypeStruct((2 * N_PAD, DH), jnp.float32),
    )(h, deg)


def _epi_body(agg_ref, hp_ref, deg_ref, b_ref, w_ref, x_ref, hpn_ref):
    dinv = lax.rsqrt(deg_ref[:, :1] + 1.0)
    xl = jnp.maximum(dinv * (agg_ref[...] + hp_ref[...]) + b_ref[...], 0.0)
    x_ref[...] = xl
    hpn_ref[...] = jnp.dot(xl, w_ref[...], preferred_element_type=jnp.float32) * dinv


def _epi(agg, hp, deg, b, Wn):
    return pl.pallas_call(
        _epi_body,
        grid=(_GRID,),
        in_specs=[
            pl.BlockSpec((_BLK, DH), lambda i: (i, 0)),
            pl.BlockSpec((_BLK, DH), lambda i: (i, 0)),
            pl.BlockSpec((_BLK, DH), lambda i: (i, 0)),
            pl.BlockSpec((1, DH), lambda i: (0, 0)),
            pl.BlockSpec((DH, DH), lambda i: (0, 0)),
        ],
        out_specs=[
            pl.BlockSpec((_BLK, DH), lambda i: (i, 0)),
            pl.BlockSpec((_BLK, DH), lambda i: (i, 0)),
        ],
        out_shape=[
            jax.ShapeDtypeStruct((2 * N_PAD, DH), jnp.float32),
            jax.ShapeDtypeStruct((2 * N_PAD, DH), jnp.float32),
        ],
    )(agg, hp, deg, b, Wn)


def _rowepi_body(agg_ref, hp_ref, deg_ref, b_ref, x_ref):
    dinv = lax.rsqrt(deg_ref[:, :1] + 1.0)
    x_ref[...] = jnp.maximum(dinv * (agg_ref[...] + hp_ref[...]) + b_ref[...], 0.0)


def _rowepi(agg, hp, deg, b):
    return pl.pallas_call(
        _rowepi_body,
        grid=(NG // _BLK,),
        in_specs=[
            pl.BlockSpec((_BLK, DH), lambda i: (i, 0)),
            pl.BlockSpec((_BLK, DH), lambda i: (i, 0)),
            pl.BlockSpec((_BLK, DH), lambda i: (i, 0)),
            pl.BlockSpec((1, DH), lambda i: (0, 0)),
        ],
        out_specs=pl.BlockSpec((_BLK, DH), lambda i: (i, 0)),
        out_shape=jax.ShapeDtypeStruct((NG, DH), jnp.float32),
    )(agg, hp, deg, b)


def _bmm_body(a0, a1, b0, b1, w0s, w1s, w0t, w1t, o_ref):
    ys = w0s[0] * a0[0] + w1s[0] * a1[0]
    yt = w0t[0] * b0[0] + w1t[0] * b1[0]
    o_ref[0, 0] = lax.dot_general(ys, yt, (((1,), (1,)), ((), ())),
                                  preferred_element_type=jnp.float32)


def _bmm(a0, a1, b0, b1, w0s, w1s, w0t, w1t):
    row_spec = pl.BlockSpec((1, J, DH), lambda l, b: (l, b, 0))
    wt_spec = pl.BlockSpec((1, J, DH), lambda l, b: (b, 0, 0))
    return pl.pallas_call(
        _bmm_body,
        grid=(3, B),
        in_specs=[row_spec, row_spec, row_spec, row_spec,
                  wt_spec, wt_spec, wt_spec, wt_spec],
        out_specs=pl.BlockSpec((1, 1, J, J), lambda l, b: (b, l, 0, 0)),
        out_shape=jax.ShapeDtypeStruct((B, 3, J, J), jnp.float32),
    )(a0, a1, b0, b1, w0s, w1s, w0t, w1t)


def _pad_edges(ei, core):
    src, dst = ei[0].astype(jnp.int32), ei[1].astype(jnp.int32)
    npad = E_PAD - src.shape[0]
    padr = (N + (jnp.arange(npad, dtype=jnp.int32) % 16))
    src_p = jnp.concatenate([src, padr])
    dst_p = jnp.concatenate([dst, padr])
    return src_p + core * N_PAD, dst_p


def _interp_geometry():
    scale = MAXN / RESHAPE
    c = (jnp.arange(RESHAPE, dtype=jnp.float32) + 0.5) * scale - 0.5
    c0 = jnp.floor(c)
    w = c - c0
    i0 = jnp.clip(c0, 0, MAXN - 1).astype(jnp.int32)
    i1 = jnp.clip(c0 + 1, 0, MAXN - 1).astype(jnp.int32)
    return i0, i1, w


def _proj_idx(batch, max_num_nodes, base, i0, i1, w):
    bd = jnp.searchsorted(batch, jnp.arange(B + 1, dtype=batch.dtype)).astype(jnp.int32)
    starts, counts = bd[:B], bd[1:] - bd[:B]
    lim = jnp.minimum(counts, max_num_nodes)
    v0 = i0[None, :] < lim[:, None]
    v1 = i1[None, :] < lim[:, None]
    spread = (jnp.arange(B, dtype=jnp.int32)[:, None] * J
              + jnp.arange(RESHAPE, dtype=jnp.int32)[None, :])
    g0 = jnp.where(v0, starts[:, None] + i0[None, :], spread) + base
    g1 = jnp.where(v1, starts[:, None] + i1[None, :], spread) + base
    w0 = jnp.where(v0, 1.0 - w[None, :], 0.0)
    w1 = jnp.where(v1, w[None, :], 0.0)
    pad2 = ((0, 0), (0, J - RESHAPE))
    g0 = jnp.pad(g0, pad2, constant_values=base)
    g1 = jnp.pad(g1, pad2, constant_values=base)
    w0 = jnp.pad(w0, pad2)
    w1 = jnp.pad(w1, pad2)
    return (g0.reshape(-1), g1.reshape(-1),
            jnp.broadcast_to(w0.reshape(B, J, 1), (B, J, DH)),
            jnp.broadcast_to(w1.reshape(B, J, 1), (B, J, DH)))


def kernel(x_s, edge_index_s, batch_s, x_t, edge_index_t, batch_t,
           W1, b1, W2, b2, W3, b3, max_num_nodes):
    f32 = jnp.float32
    x_both = jnp.zeros((2 * N_PAD, DIN), f32)
    x_both = x_both.at[:N].set(x_s).at[N_PAD:N_PAD + N].set(x_t)

    src_s, dst_s = _pad_edges(edge_index_s, 0)
    src_t, dst_t = _pad_edges(edge_index_t, 1)
    src_a = jnp.stack([src_s, src_t]).reshape(2, NSUB, NCHUNK, 1, CHUNK)
    dst_a = jnp.stack([dst_s, dst_t]).reshape(2, NSUB, NCHUNK, 1, CHUNK)
    eidx = jnp.concatenate([src_a, dst_a], axis=3)
    dst_g = dst_a.reshape(2, NSUB, NCHUNK, CHUNK)

    ones128 = jnp.ones((CHUNK, DH), f32)
    zeros128 = jnp.zeros((N_PAD, DH), f32)

    h1 = _mmh(x_both, W1)
    deg = _sc_deg(dst_g, ones128, zeros128).reshape(2 * N_PAD, DH)

    b1r = b1.reshape(1, DH)
    b2r = b2.reshape(1, DH)
    b3r = b3.reshape(1, DH)

    hp1 = _scale(h1, deg)
    agg1 = _sc_conv(hp1, eidx, zeros128).reshape(2 * N_PAD, DH)
    x1, hp2 = _epi(agg1, hp1, deg, b1r, W2)
    agg2 = _sc_conv(hp2, eidx, zeros128).reshape(2 * N_PAD, DH)
    x2, hp3 = _epi(agg2, hp2, deg, b2r, W3)

    i0, i1, w = _interp_geometry()
    g0s, g1s, w0s, w1s = _proj_idx(batch_s, max_num_nodes, 0, i0, i1, w)
    g0t, g1t, w0t, w1t = _proj_idx(batch_t, max_num_nodes, N_PAD, i0, i1, w)
    gidx = jnp.concatenate([g0s, g1s, g0t, g1t]).reshape(32, PERW)
    aggidx = gidx - jnp.where(jnp.arange(32) < NSUB, 0, N_PAD)[:, None].astype(jnp.int32)

    rows5 = _sc_conv3(hp3, eidx, zeros128, x1, x2, deg, gidx, aggidx)
    x3rows = _rowepi(rows5[4], rows5[2], rows5[3], b3r)
    rows = jnp.stack([rows5[0], rows5[1], x3rows])
    nbj = B * J
    a0 = rows[:, 0 * nbj:1 * nbj]
    a1 = rows[:, 1 * nbj:2 * nbj]
    c0 = rows[:, 2 * nbj:3 * nbj]
    c1 = rows[:, 3 * nbj:4 * nbj]

    out_full = _bmm(a0, a1, c0, c1, w0s, w1s, w0t, w1t)
    return out_full[:, :, :RESHAPE, :RESHAPE]
